# initial kernel scaffold (unmeasured)
import jax
import jax.numpy as jnp
from jax import lax
from jax.experimental import pallas as pl
from jax.experimental.pallas import tpu as pltpu


def kernel(
    x,
):
    def body(*refs):
        pass

    out_shape = jax.ShapeDtypeStruct(..., jnp.float32)
    return pl.pallas_call(body, out_shape=out_shape)(...)


# baseline (device time: 49568 ns/iter reference)
import jax
import jax.numpy as jnp
from jax import lax
from jax.experimental import pallas as pl
from jax.experimental.pallas import tpu as pltpu

N_DEV = 4
N_HOPS = N_DEV - 1


def kernel(x):
    m_per, n = x.shape
    m_c = m_per // N_DEV

    def body(x_ref, out_ref, rs_send, rs_recv, send_sems, recv_sems):
        my = lax.axis_index("i")
        left = lax.rem(my + N_DEV - 1, N_DEV)
        right = lax.rem(my + 1, N_DEV)

        barrier_sem = pltpu.get_barrier_semaphore()
        for nbr in (left, right):
            pl.semaphore_signal(
                barrier_sem, inc=1,
                device_id=(nbr,), device_id_type=pl.DeviceIdType.MESH,
            )
        pl.semaphore_wait(barrier_sem, 2)

        rs_send[0, :, :] = x_ref[pl.ds(my * m_c, m_c), :]
        for s in range(N_HOPS):
            rdma = pltpu.make_async_remote_copy(
                src_ref=rs_send.at[s],
                dst_ref=rs_recv.at[s],
                send_sem=send_sems.at[s],
                recv_sem=recv_sems.at[s],
                device_id=(right,),
                device_id_type=pl.DeviceIdType.MESH,
            )
            rdma.start()
            rdma.wait()

            c_recv = lax.rem(my + N_DEV - s - 1, N_DEV)
            partial = rs_recv[s, :, :] + x_ref[pl.ds(c_recv * m_c, m_c), :]
            if s < N_HOPS - 1:
                rs_send[s + 1, :, :] = partial
            else:
                out_ref[pl.ds(right * m_c, m_c), :] = partial

        for t in range(N_HOPS):
            h = N_HOPS + t
            c = lax.rem(my + 1 - t + N_DEV, N_DEV)
            rdma = pltpu.make_async_remote_copy(
                src_ref=out_ref.at[pl.ds(c * m_c, m_c), :],
                dst_ref=out_ref.at[pl.ds(c * m_c, m_c), :],
                send_sem=send_sems.at[h],
                recv_sem=recv_sems.at[h],
                device_id=(right,),
                device_id_type=pl.DeviceIdType.MESH,
            )
            rdma.start()
            rdma.wait()

    return pl.pallas_call(
        body,
        out_shape=jax.ShapeDtypeStruct((m_per, n), x.dtype),
        in_specs=[pl.BlockSpec(memory_space=pltpu.VMEM)],
        out_specs=pl.BlockSpec(memory_space=pltpu.VMEM),
        scratch_shapes=[
            pltpu.VMEM((N_HOPS, m_c, n), x.dtype),
            pltpu.VMEM((N_HOPS, m_c, n), x.dtype),
            pltpu.SemaphoreType.DMA((2 * N_HOPS,)),
            pltpu.SemaphoreType.DMA((2 * N_HOPS,)),
        ],
        compiler_params=pltpu.CompilerParams(collective_id=0),
    )(x)


# device time: 26079 ns/iter; 1.9007x vs baseline; 1.9007x over previous
import jax
import jax.numpy as jnp
from jax import lax
from jax.experimental import pallas as pl
from jax.experimental.pallas import tpu as pltpu

N_DEV = 4
N_HOPS = N_DEV - 1
N_SUB = 2


def kernel(x):
    m_per, n = x.shape
    m_c = m_per // N_DEV
    m_s = m_c // N_SUB
    n_h = n // 2

    def body(x_ref, out_ref, sendbuf, recvbuf, send_sems, recv_sems):
        my = lax.axis_index("i")
        left = lax.rem(my + N_DEV - 1, N_DEV)
        right = lax.rem(my + 1, N_DEV)

        nbr_out = (right, left)
        col0 = (0, n_h)

        def rs_chunk(d, s):
            if d == 0:
                return lax.rem(my + N_DEV - s, N_DEV)
            return lax.rem(my + s, N_DEV)

        def ag_chunk(d, t):
            if d == 0:
                return lax.rem(my + 1 - t + N_DEV, N_DEV)
            return lax.rem(my - 1 + t + N_DEV, N_DEV)

        barrier_sem = pltpu.get_barrier_semaphore()
        for nbr in (left, right):
            pl.semaphore_signal(
                barrier_sem, inc=1,
                device_id=(nbr,), device_id_type=pl.DeviceIdType.MESH,
            )
        pl.semaphore_wait(barrier_sem, 2)

        pending_sends = []

        def start_rdma(src, dst, h, k, d, target):
            rdma = pltpu.make_async_remote_copy(
                src_ref=src, dst_ref=dst,
                send_sem=send_sems.at[h, k, d],
                recv_sem=recv_sems.at[h, k, d],
                device_id=(target,),
                device_id_type=pl.DeviceIdType.MESH,
            )
            rdma.start()
            pending_sends.append(rdma)
            return rdma

        rs = {}
        for k in range(N_SUB):
            for d in range(2):
                c = rs_chunk(d, 0)
                rs[(d, 0, k)] = start_rdma(
                    x_ref.at[pl.ds(c * m_c + k * m_s, m_s),
                             pl.ds(col0[d], n_h)],
                    recvbuf.at[d, 0, pl.ds(k * m_s, m_s), :],
                    0, k, d, nbr_out[d],
                )

        for s in range(N_HOPS):
            for k in range(N_SUB):
                for d in range(2):
                    rs[(d, s, k)].wait_recv()
                    c_recv = rs_chunk(d, s + 1)
                    row = pl.ds(c_recv * m_c + k * m_s, m_s)
                    sub = pl.ds(k * m_s, m_s)
                    partial = (recvbuf[d, s, sub, :]
                               + x_ref[row, pl.ds(col0[d], n_h)])
                    if s < N_HOPS - 1:
                        sendbuf[d, s + 1, sub, :] = partial
                        rs[(d, s + 1, k)] = start_rdma(
                            sendbuf.at[d, s + 1, sub, :],
                            recvbuf.at[d, s + 1, sub, :],
                            s + 1, k, d, nbr_out[d],
                        )
                    else:
                        out_ref[row, pl.ds(col0[d], n_h)] = partial
                        ag_slice = out_ref.at[row, pl.ds(col0[d], n_h)]
                        rs[(d, N_HOPS, k)] = start_rdma(
                            ag_slice, ag_slice, N_HOPS, k, d, nbr_out[d],
                        )

        ag = {(d, 0, k): rs[(d, N_HOPS, k)]
              for d in range(2) for k in range(N_SUB)}
        for t in range(N_HOPS):
            for k in range(N_SUB):
                for d in range(2):
                    ag[(d, t, k)].wait_recv()
                    if t < N_HOPS - 1:
                        c = ag_chunk(d, t + 1)
                        sl = out_ref.at[pl.ds(c * m_c + k * m_s, m_s),
                                        pl.ds(col0[d], n_h)]
                        ag[(d, t + 1, k)] = start_rdma(
                            sl, sl, N_HOPS + t + 1, k, d, nbr_out[d],
                        )

        for rdma in pending_sends:
            rdma.wait_send()

    n_sem = 2 * N_HOPS
    return pl.pallas_call(
        body,
        out_shape=jax.ShapeDtypeStruct((m_per, n), x.dtype),
        in_specs=[pl.BlockSpec(memory_space=pltpu.VMEM)],
        out_specs=pl.BlockSpec(memory_space=pltpu.VMEM),
        scratch_shapes=[
            pltpu.VMEM((2, N_HOPS, m_c, n // 2), x.dtype),
            pltpu.VMEM((2, N_HOPS, m_c, n // 2), x.dtype),
            pltpu.SemaphoreType.DMA((n_sem, N_SUB, 2)),
            pltpu.SemaphoreType.DMA((n_sem, N_SUB, 2)),
        ],
        compiler_params=pltpu.CompilerParams(collective_id=0),
    )(x)


# device time: 24386 ns/iter; 2.0326x vs baseline; 1.0694x over previous
import jax
import jax.numpy as jnp
from jax import lax
from jax.experimental import pallas as pl
from jax.experimental.pallas import tpu as pltpu

N_DEV = 4
N_HOPS = N_DEV - 1
N_SUB = 4


def kernel(x):
    m_per, n = x.shape
    m_c = m_per // N_DEV
    m_s = m_c // N_SUB
    n_h = n // 2

    def body(x_ref, out_ref, sendbuf, recvbuf, send_sems, recv_sems):
        my = lax.axis_index("i")
        left = lax.rem(my + N_DEV - 1, N_DEV)
        right = lax.rem(my + 1, N_DEV)

        nbr_out = (right, left)
        col0 = (0, n_h)

        def rs_chunk(d, s):
            if d == 0:
                return lax.rem(my + N_DEV - s, N_DEV)
            return lax.rem(my + s, N_DEV)

        def ag_chunk(d, t):
            if d == 0:
                return lax.rem(my + 1 - t + N_DEV, N_DEV)
            return lax.rem(my - 1 + t + N_DEV, N_DEV)

        barrier_sem = pltpu.get_barrier_semaphore()
        for nbr in (left, right):
            pl.semaphore_signal(
                barrier_sem, inc=1,
                device_id=(nbr,), device_id_type=pl.DeviceIdType.MESH,
            )
        pl.semaphore_wait(barrier_sem, 2)

        pending_sends = []

        def start_rdma(src, dst, h, k, d, target):
            rdma = pltpu.make_async_remote_copy(
                src_ref=src, dst_ref=dst,
                send_sem=send_sems.at[h, k, d],
                recv_sem=recv_sems.at[h, k, d],
                device_id=(target,),
                device_id_type=pl.DeviceIdType.MESH,
            )
            rdma.start()
            pending_sends.append(rdma)
            return rdma

        rs = {}
        for k in range(N_SUB):
            for d in range(2):
                c = rs_chunk(d, 0)
                rs[(d, 0, k)] = start_rdma(
                    x_ref.at[pl.ds(c * m_c + k * m_s, m_s),
                             pl.ds(col0[d], n_h)],
                    recvbuf.at[d, 0, pl.ds(k * m_s, m_s), :],
                    0, k, d, nbr_out[d],
                )

        for s in range(N_HOPS):
            for k in range(N_SUB):
                for d in range(2):
                    rs[(d, s, k)].wait_recv()
                    c_recv = rs_chunk(d, s + 1)
                    row = pl.ds(c_recv * m_c + k * m_s, m_s)
                    sub = pl.ds(k * m_s, m_s)
                    partial = (recvbuf[d, s, sub, :]
                               + x_ref[row, pl.ds(col0[d], n_h)])
                    if s < N_HOPS - 1:
                        sendbuf[d, s + 1, sub, :] = partial
                        rs[(d, s + 1, k)] = start_rdma(
                            sendbuf.at[d, s + 1, sub, :],
                            recvbuf.at[d, s + 1, sub, :],
                            s + 1, k, d, nbr_out[d],
                        )
                    else:
                        out_ref[row, pl.ds(col0[d], n_h)] = partial
                        ag_slice = out_ref.at[row, pl.ds(col0[d], n_h)]
                        rs[(d, N_HOPS, k)] = start_rdma(
                            ag_slice, ag_slice, N_HOPS, k, d, nbr_out[d],
                        )

        ag = {(d, 0, k): rs[(d, N_HOPS, k)]
              for d in range(2) for k in range(N_SUB)}
        for t in range(N_HOPS):
            for k in range(N_SUB):
                for d in range(2):
                    ag[(d, t, k)].wait_recv()
                    if t < N_HOPS - 1:
                        c = ag_chunk(d, t + 1)
                        sl = out_ref.at[pl.ds(c * m_c + k * m_s, m_s),
                                        pl.ds(col0[d], n_h)]
                        ag[(d, t + 1, k)] = start_rdma(
                            sl, sl, N_HOPS + t + 1, k, d, nbr_out[d],
                        )

        for rdma in pending_sends:
            rdma.wait_send()

    n_sem = 2 * N_HOPS
    return pl.pallas_call(
        body,
        out_shape=jax.ShapeDtypeStruct((m_per, n), x.dtype),
        in_specs=[pl.BlockSpec(memory_space=pltpu.VMEM)],
        out_specs=pl.BlockSpec(memory_space=pltpu.VMEM),
        scratch_shapes=[
            pltpu.VMEM((2, N_HOPS, m_c, n // 2), x.dtype),
            pltpu.VMEM((2, N_HOPS, m_c, n // 2), x.dtype),
            pltpu.SemaphoreType.DMA((n_sem, N_SUB, 2)),
            pltpu.SemaphoreType.DMA((n_sem, N_SUB, 2)),
        ],
        compiler_params=pltpu.CompilerParams(collective_id=0),
    )(x)
